# baseline (device time: 32055 ns/iter reference)
import jax
import jax.numpy as jnp
from jax import lax
from jax.experimental import pallas as pl
from jax.experimental.pallas import tpu as pltpu

N_DEV = 4
B, Sq, Hq, Hkv, Dh = 2, 256, 8, 2, 64
G = Hq // Hkv
SCALE = 0.125


def kernel(x, Wq, Wo, K_ext, V_ext):
    c = K_ext.shape[1]

    def body(x_hbm, wq_hbm, wo_hbm, kt_hbm, vt_hbm, out_ref,
             x_ref, wq_ref, wo_ref, kt_ref, vt_ref,
             kbuf, vbuf, in_sems, k_send, k_recv, v_send, v_recv):
        my = lax.axis_index("i")

        in_dmas = [
            pltpu.make_async_copy(src, dst, in_sems.at[i])
            for i, (src, dst) in enumerate([
                (kt_hbm, kt_ref), (vt_hbm, vt_ref), (x_hbm, x_ref),
                (wq_hbm, wq_ref), (wo_hbm, wo_ref)])
        ]
        for dma in in_dmas:
            dma.start()

        barrier = pltpu.get_barrier_semaphore()
        for d in range(1, N_DEV):
            pl.semaphore_signal(barrier, inc=1, device_id=((my + d) % N_DEV,),
                                device_id_type=pl.DeviceIdType.MESH)
        pl.semaphore_wait(barrier, N_DEV - 1)
        in_dmas[0].wait()
        in_dmas[1].wait()

        c2 = c // 2
        ones_col = jnp.ones((c2, 1), jnp.bfloat16)
        for b in range(B):
            for kvh in range(Hkv):
                kb = kt_ref[b, kvh].astype(jnp.bfloat16)
                vb = vt_ref[b, kvh].astype(jnp.bfloat16)
                for h in range(2):
                    sl = slice(h * c2, (h + 1) * c2)
                    kbuf[0, h, b, kvh] = kb[:, sl]
                    vbuf[0, h, b, kvh] = jnp.concatenate(
                        [jnp.transpose(vb[:, sl]), ones_col], axis=1)

        krd, vrd = {}, {}
        for d in (1, 3, 2):
            tgt = (my + d) % N_DEV
            for h in range(2):
                krd[d, h] = pltpu.make_async_remote_copy(
                    src_ref=kbuf.at[0, h], dst_ref=kbuf.at[N_DEV - d, h],
                    send_sem=k_send.at[(d - 1) * 2 + h],
                    recv_sem=k_recv.at[(N_DEV - 1 - d) * 2 + h],
                    device_id=(tgt,), device_id_type=pl.DeviceIdType.MESH)
                vrd[d, h] = pltpu.make_async_remote_copy(
                    src_ref=vbuf.at[0, h], dst_ref=vbuf.at[N_DEV - d, h],
                    send_sem=v_send.at[(d - 1) * 2 + h],
                    recv_sem=v_recv.at[(N_DEV - 1 - d) * 2 + h],
                    device_id=(tgt,), device_id_type=pl.DeviceIdType.MESH)
                krd[d, h].start()
                vrd[d, h].start()

        in_dmas[2].wait()
        in_dmas[3].wait()
        x2d = jnp.reshape(x_ref[...], (B * Sq, x_ref.shape[2]))
        qall = jnp.dot(x2d.astype(jnp.bfloat16),
                       wq_ref[...].astype(jnp.bfloat16),
                       preferred_element_type=jnp.float32)
        qallh = (qall * SCALE).astype(jnp.bfloat16)
        qg = []
        for b in range(B):
            qbh = qallh[b * Sq:(b + 1) * Sq, :]
            qg.append([
                jnp.concatenate(
                    [qbh[:, (kvh * G + g) * Dh:(kvh * G + g + 1) * Dh]
                     for g in range(G)], axis=0)
                for kvh in range(Hkv)
            ])

        acc = [[jnp.zeros((G * Sq, Dh + 1), jnp.float32)
                for _ in range(Hkv)] for _ in range(B)]

        def fold_half(slot, h, k_wait=None, v_wait=None):
            if k_wait is not None:
                k_wait.wait_recv()
            ps = []
            for b in range(B):
                for kvh in range(Hkv):
                    kc = kbuf[slot, h, b, kvh]
                    s = jnp.dot(qg[b][kvh], kc,
                                preferred_element_type=jnp.float32)
                    ps.append(jnp.exp(s).astype(jnp.bfloat16))
            if v_wait is not None:
                v_wait.wait_recv()
            for b in range(B):
                for kvh in range(Hkv):
                    vc = vbuf[slot, h, b, kvh]
                    acc[b][kvh] = acc[b][kvh] + jnp.dot(
                        ps[b * Hkv + kvh], vc,
                        preferred_element_type=jnp.float32)

        fold_half(0, 0)
        fold_half(0, 1)
        for slot, d in ((3, 1), (1, 3), (2, 2)):
            for h in range(2):
                fold_half(slot, h, k_wait=krd[d, h], v_wait=vrd[d, h])

        in_dmas[4].wait()
        rows = []
        for b in range(B):
            heads = []
            for h in range(Hq):
                kvh, g = h // G, h % G
                blk = acc[b][kvh][g * Sq:(g + 1) * Sq, :]
                heads.append(blk[:, :Dh] / blk[:, Dh:Dh + 1])
            rows.append(jnp.concatenate(heads, axis=1))
        ob = jnp.concatenate(rows, axis=0)
        res = jnp.dot(ob.astype(jnp.bfloat16),
                      wo_ref[...].astype(jnp.bfloat16),
                      preferred_element_type=jnp.float32)
        out_ref[...] = jnp.reshape(res, (B, Sq, res.shape[1]))

        for d in range(1, N_DEV):
            for h in range(2):
                krd[d, h].wait_send()
                vrd[d, h].wait_send()

    return pl.pallas_call(
        body,
        out_shape=jax.ShapeDtypeStruct((B, Sq, Wo.shape[1]), jnp.float32),
        in_specs=[pl.BlockSpec(memory_space=pl.ANY)] * 5,
        out_specs=pl.BlockSpec(memory_space=pltpu.VMEM),
        scratch_shapes=[
            pltpu.VMEM(x.shape, jnp.float32),
            pltpu.VMEM(Wq.shape, jnp.float32),
            pltpu.VMEM(Wo.shape, jnp.float32),
            pltpu.VMEM((B, Hkv, Dh, c), jnp.float32),
            pltpu.VMEM((B, Hkv, Dh, c), jnp.float32),
            pltpu.VMEM((N_DEV, 2, B, Hkv, Dh, c // 2), jnp.bfloat16),
            pltpu.VMEM((N_DEV, 2, B, Hkv, c // 2, Dh + 1), jnp.bfloat16),
            pltpu.SemaphoreType.DMA((5,)),
            pltpu.SemaphoreType.DMA(((N_DEV - 1) * 2,)),
            pltpu.SemaphoreType.DMA(((N_DEV - 1) * 2,)),
            pltpu.SemaphoreType.DMA(((N_DEV - 1) * 2,)),
            pltpu.SemaphoreType.DMA(((N_DEV - 1) * 2,)),
        ],
        compiler_params=pltpu.CompilerParams(collective_id=0),
    )(x, Wq, Wo,
      jnp.transpose(K_ext, (0, 2, 3, 1)), jnp.transpose(V_ext, (0, 2, 3, 1)))


# device time: 31891 ns/iter; 1.0051x vs baseline; 1.0051x over previous
import jax
import jax.numpy as jnp
from jax import lax
from jax.experimental import pallas as pl
from jax.experimental.pallas import tpu as pltpu

N_DEV = 4
B, Sq, Hq, Hkv, Dh = 2, 256, 8, 2, 64
G = Hq // Hkv
SCALE = 0.125


def kernel(x, Wq, Wo, K_ext, V_ext):
    c = K_ext.shape[1]

    def body(x_ref, wq_ref, wo_ref, kt_ref, vt_ref, out_ref,
             kbuf, vbuf, k_send, k_recv, v_send, v_recv):
        my = lax.axis_index("i")

        barrier = pltpu.get_barrier_semaphore()
        for d in range(1, N_DEV):
            pl.semaphore_signal(barrier, inc=1, device_id=((my + d) % N_DEV,),
                                device_id_type=pl.DeviceIdType.MESH)
        pl.semaphore_wait(barrier, N_DEV - 1)

        c2 = c // 2
        ones_col = jnp.ones((c2, 1), jnp.bfloat16)
        for b in range(B):
            for kvh in range(Hkv):
                kb = kt_ref[b, kvh].astype(jnp.bfloat16)
                vb = vt_ref[b, kvh].astype(jnp.bfloat16)
                for h in range(2):
                    sl = slice(h * c2, (h + 1) * c2)
                    kbuf[0, h, b, kvh] = kb[:, sl]
                    vbuf[0, h, b, kvh] = jnp.concatenate(
                        [jnp.transpose(vb[:, sl]), ones_col], axis=1)

        krd, vrd = {}, {}
        for d in (1, 3, 2):
            tgt = (my + d) % N_DEV
            for h in range(2):
                krd[d, h] = pltpu.make_async_remote_copy(
                    src_ref=kbuf.at[0, h], dst_ref=kbuf.at[N_DEV - d, h],
                    send_sem=k_send.at[(d - 1) * 2 + h],
                    recv_sem=k_recv.at[(N_DEV - 1 - d) * 2 + h],
                    device_id=(tgt,), device_id_type=pl.DeviceIdType.MESH)
                vrd[d, h] = pltpu.make_async_remote_copy(
                    src_ref=vbuf.at[0, h], dst_ref=vbuf.at[N_DEV - d, h],
                    send_sem=v_send.at[(d - 1) * 2 + h],
                    recv_sem=v_recv.at[(N_DEV - 1 - d) * 2 + h],
                    device_id=(tgt,), device_id_type=pl.DeviceIdType.MESH)
                krd[d, h].start()
                vrd[d, h].start()

        x2d = jnp.reshape(x_ref[...], (B * Sq, x_ref.shape[2]))
        qall = jnp.dot(x2d.astype(jnp.bfloat16),
                       wq_ref[...].astype(jnp.bfloat16),
                       preferred_element_type=jnp.float32)
        qallh = (qall * (SCALE * 1.4426950408889634)).astype(jnp.bfloat16)
        qg = []
        for b in range(B):
            qbh = qallh[b * Sq:(b + 1) * Sq, :]
            qg.append([
                jnp.concatenate(
                    [qbh[:, (kvh * G + g) * Dh:(kvh * G + g + 1) * Dh]
                     for g in range(G)], axis=0)
                for kvh in range(Hkv)
            ])

        acc = [[jnp.zeros((G * Sq, Dh + 1), jnp.float32)
                for _ in range(Hkv)] for _ in range(B)]

        def fold_half(slot, h, k_wait=None, v_wait=None):
            if k_wait is not None:
                k_wait.wait_recv()
            ps = []
            for b in range(B):
                for kvh in range(Hkv):
                    kc = kbuf[slot, h, b, kvh]
                    s = jnp.dot(qg[b][kvh], kc,
                                preferred_element_type=jnp.float32)
                    ps.append(jnp.exp2(s).astype(jnp.bfloat16))
            if v_wait is not None:
                v_wait.wait_recv()
            for b in range(B):
                for kvh in range(Hkv):
                    vc = vbuf[slot, h, b, kvh]
                    acc[b][kvh] = acc[b][kvh] + jnp.dot(
                        ps[b * Hkv + kvh], vc,
                        preferred_element_type=jnp.float32)

        fold_half(0, 0)
        fold_half(0, 1)
        for slot, d in ((3, 1), (1, 3), (2, 2)):
            for h in range(2):
                fold_half(slot, h, k_wait=krd[d, h], v_wait=vrd[d, h])

        rows = []
        for b in range(B):
            heads = []
            for h in range(Hq):
                kvh, g = h // G, h % G
                blk = acc[b][kvh][g * Sq:(g + 1) * Sq, :]
                heads.append(blk[:, :Dh] / blk[:, Dh:Dh + 1])
            rows.append(jnp.concatenate(heads, axis=1))
        ob = jnp.concatenate(rows, axis=0)
        res = jnp.dot(ob.astype(jnp.bfloat16),
                      wo_ref[...].astype(jnp.bfloat16),
                      preferred_element_type=jnp.float32)
        out_ref[...] = jnp.reshape(res, (B, Sq, res.shape[1]))

        for d in range(1, N_DEV):
            for h in range(2):
                krd[d, h].wait_send()
                vrd[d, h].wait_send()

    return pl.pallas_call(
        body,
        out_shape=jax.ShapeDtypeStruct((B, Sq, Wo.shape[1]), jnp.float32),
        in_specs=[pl.BlockSpec(memory_space=pltpu.VMEM)] * 5,
        out_specs=pl.BlockSpec(memory_space=pltpu.VMEM),
        scratch_shapes=[
            pltpu.VMEM((N_DEV, 2, B, Hkv, Dh, c // 2), jnp.bfloat16),
            pltpu.VMEM((N_DEV, 2, B, Hkv, c // 2, Dh + 1), jnp.bfloat16),
            pltpu.SemaphoreType.DMA(((N_DEV - 1) * 2,)),
            pltpu.SemaphoreType.DMA(((N_DEV - 1) * 2,)),
            pltpu.SemaphoreType.DMA(((N_DEV - 1) * 2,)),
            pltpu.SemaphoreType.DMA(((N_DEV - 1) * 2,)),
        ],
        compiler_params=pltpu.CompilerParams(collective_id=0),
    )(x, Wq, Wo,
      jnp.transpose(K_ext, (0, 2, 3, 1)), jnp.transpose(V_ext, (0, 2, 3, 1)))


# device time: 27167 ns/iter; 1.1799x vs baseline; 1.1739x over previous
import jax
import jax.numpy as jnp
from jax import lax
from jax.experimental import pallas as pl
from jax.experimental.pallas import tpu as pltpu

N_DEV = 4
B, Sq, Hq, Hkv, Dh = 2, 256, 8, 2, 64
G = Hq // Hkv
SCALE = 0.125


def kernel(x, Wq, Wo, K_ext, V_ext):
    c = K_ext.shape[1]

    def body(x_ref, wq_ref, wo_ref, kt_ref, vt_ref, out_ref,
             kbuf, vbuf, k_send, k_recv, v_send, v_recv):
        my = lax.axis_index("i")

        barrier = pltpu.get_barrier_semaphore()
        for d in range(1, N_DEV):
            pl.semaphore_signal(barrier, inc=1, device_id=((my + d) % N_DEV,),
                                device_id_type=pl.DeviceIdType.MESH)
        pl.semaphore_wait(barrier, N_DEV - 1)

        c2 = c // 2
        ones_row = jnp.ones((1, c2), jnp.bfloat16)
        for b in range(B):
            for kvh in range(Hkv):
                kb = kt_ref[b, kvh].astype(jnp.bfloat16)
                vb = vt_ref[b, kvh].astype(jnp.bfloat16)
                for h in range(2):
                    sl = slice(h * c2, (h + 1) * c2)
                    kbuf[0, h, b, kvh] = kb[:, sl]
                    vbuf[0, h, b, kvh] = jnp.concatenate(
                        [vb[:, sl], ones_row], axis=0)

        krd, vrd = {}, {}
        for d in (1, 3, 2):
            tgt = (my + d) % N_DEV
            for h in range(2):
                krd[d, h] = pltpu.make_async_remote_copy(
                    src_ref=kbuf.at[0, h], dst_ref=kbuf.at[N_DEV - d, h],
                    send_sem=k_send.at[(d - 1) * 2 + h],
                    recv_sem=k_recv.at[(N_DEV - 1 - d) * 2 + h],
                    device_id=(tgt,), device_id_type=pl.DeviceIdType.MESH)
                vrd[d, h] = pltpu.make_async_remote_copy(
                    src_ref=vbuf.at[0, h], dst_ref=vbuf.at[N_DEV - d, h],
                    send_sem=v_send.at[(d - 1) * 2 + h],
                    recv_sem=v_recv.at[(N_DEV - 1 - d) * 2 + h],
                    device_id=(tgt,), device_id_type=pl.DeviceIdType.MESH)
                krd[d, h].start()
                vrd[d, h].start()

        x2d = jnp.reshape(x_ref[...], (B * Sq, x_ref.shape[2]))
        qall = jnp.dot(x2d.astype(jnp.bfloat16),
                       wq_ref[...].astype(jnp.bfloat16),
                       preferred_element_type=jnp.float32)
        qallh = (qall * (SCALE * 1.4426950408889634)).astype(jnp.bfloat16)
        qg = []
        for b in range(B):
            qbh = qallh[b * Sq:(b + 1) * Sq, :]
            qg.append([
                jnp.concatenate(
                    [qbh[:, (kvh * G + g) * Dh:(kvh * G + g + 1) * Dh]
                     for g in range(G)], axis=0)
                for kvh in range(Hkv)
            ])

        acc = [[jnp.zeros((G * Sq, Dh + 1), jnp.float32)
                for _ in range(Hkv)] for _ in range(B)]

        def fold_half(slot, h, k_wait=None, v_wait=None):
            if k_wait is not None:
                k_wait.wait_recv()
            ps = []
            for b in range(B):
                for kvh in range(Hkv):
                    kc = kbuf[slot, h, b, kvh]
                    s = jnp.dot(qg[b][kvh], kc,
                                preferred_element_type=jnp.float32)
                    ps.append(jnp.exp2(s).astype(jnp.bfloat16))
            if v_wait is not None:
                v_wait.wait_recv()
            for b in range(B):
                for kvh in range(Hkv):
                    vc = vbuf[slot, h, b, kvh]
                    acc[b][kvh] = acc[b][kvh] + lax.dot_general(
                        ps[b * Hkv + kvh], vc, (((1,), (1,)), ((), ())),
                        preferred_element_type=jnp.float32)

        fold_half(0, 0)
        fold_half(0, 1)
        for slot, d in ((3, 1), (1, 3), (2, 2)):
            for h in range(2):
                fold_half(slot, h, k_wait=krd[d, h], v_wait=vrd[d, h])

        rows = []
        for b in range(B):
            heads = []
            for h in range(Hq):
                kvh, g = h // G, h % G
                blk = acc[b][kvh][g * Sq:(g + 1) * Sq, :]
                heads.append(blk[:, :Dh] / blk[:, Dh:Dh + 1])
            rows.append(jnp.concatenate(heads, axis=1))
        ob = jnp.concatenate(rows, axis=0)
        res = jnp.dot(ob.astype(jnp.bfloat16),
                      wo_ref[...].astype(jnp.bfloat16),
                      preferred_element_type=jnp.float32)
        out_ref[...] = jnp.reshape(res, (B, Sq, res.shape[1]))

        for d in range(1, N_DEV):
            for h in range(2):
                krd[d, h].wait_send()
                vrd[d, h].wait_send()

    return pl.pallas_call(
        body,
        out_shape=jax.ShapeDtypeStruct((B, Sq, Wo.shape[1]), jnp.float32),
        in_specs=[pl.BlockSpec(memory_space=pltpu.VMEM)] * 5,
        out_specs=pl.BlockSpec(memory_space=pltpu.VMEM),
        scratch_shapes=[
            pltpu.VMEM((N_DEV, 2, B, Hkv, Dh, c // 2), jnp.bfloat16),
            pltpu.VMEM((N_DEV, 2, B, Hkv, Dh + 1, c // 2), jnp.bfloat16),
            pltpu.SemaphoreType.DMA(((N_DEV - 1) * 2,)),
            pltpu.SemaphoreType.DMA(((N_DEV - 1) * 2,)),
            pltpu.SemaphoreType.DMA(((N_DEV - 1) * 2,)),
            pltpu.SemaphoreType.DMA(((N_DEV - 1) * 2,)),
        ],
        compiler_params=pltpu.CompilerParams(collective_id=0),
    )(x, Wq, Wo,
      jnp.transpose(K_ext, (0, 2, 3, 1)), jnp.transpose(V_ext, (0, 2, 3, 1)))
